# baseline (speedup 1.0000x reference)
import jax
import jax.numpy as jnp
from jax import lax
from jax.experimental import pallas as pl
from jax.experimental.pallas import tpu as pltpu, tpu_sc as plsc

N_TOK = 100000
N_ELEM = 100
N_ANUM = 120
NC, NS = 2, 16
NW = NC * NS
CHUNK = 128
NFULL = N_TOK // CHUNK
TAIL = N_TOK - NFULL * CHUNK
KMAX = -(-NFULL // NW)
GROUPS = CHUNK // 16
TGROUPS = TAIL // 16
TAIL_W = NFULL - (KMAX - 1) * NW


NBUF = 4


def _body(elements_hbm, eidx_hbm, eye_hbm, out_hbm,
          eidx_v, eye_v, ebuf0, ebuf1, ebuf2, ebuf3, tbuf, idxs,
          blk0, blk1, blk2, blk3, tailblk,
          esem0, esem1, esem2, esem3, osem0, osem1, osem2, osem3):
    ebuf = (ebuf0, ebuf1, ebuf2, ebuf3)
    blk = (blk0, blk1, blk2, blk3)
    cid = lax.axis_index("c")
    sid = lax.axis_index("s")
    wid = sid * NC + cid

    pltpu.sync_copy(eidx_hbm, eidx_v.at[pl.ds(0, N_ANUM)])
    pltpu.sync_copy(eye_hbm, eye_v)

    zeros16 = jnp.zeros((16,), jnp.float32)

    for buf, width in ((blk0, CHUNK), (blk1, CHUNK), (blk2, CHUNK),
                       (blk3, CHUNK), (tailblk, TAIL)):
        @pl.loop(0, N_ELEM)
        def _(r, buf=buf, width=width):
            for c in range(0, width, 16):
                buf[r, pl.ds(c, 16)] = zeros16

    lane = lax.broadcasted_iota(jnp.int32, (16,), 0)
    esem = (esem0, esem1, esem2, esem3)
    osem = (osem0, osem1, osem2, osem3)

    def elems_in(k, b):
        base = (wid + NW * k) * CHUNK
        return pltpu.make_async_copy(
            elements_hbm.at[pl.ds(base, CHUNK)], ebuf[b], esem[b]
        )

    def blk_out(k, b):
        base = (wid + NW * k) * CHUNK
        return pltpu.make_async_copy(
            blk[b], out_hbm.at[:, pl.ds(base, CHUNK)], osem[b]
        )

    for kp in range(NBUF):
        elems_in(kp, kp).start()

    def chunk_body(k, b):
        g = wid + NW * k

        @pl.when(k >= NBUF)
        def _():
            blk_out(k - NBUF, b).wait()
            for j in range(GROUPS):
                idx_g = idxs[b, j, :]
                plsc.store_scatter(blk[b], [idx_g, j * 16 + lane], zeros16)

        @pl.when(g < NFULL)
        def _():
            elems_in(k, b).wait()
            for j in range(GROUPS):
                elems_g = ebuf[b][pl.ds(j * 16, 16)]
                idx_g = plsc.load_gather(eidx_v, [elems_g])
                val_g = plsc.load_gather(eye_v, [idx_g, idx_g])
                plsc.store_scatter(blk[b], [idx_g, j * 16 + lane], val_g)
                idxs[b, j, :] = idx_g
            blk_out(k, b).start()

            @pl.when(g + NBUF * NW < NFULL)
            def _():
                elems_in(k + NBUF, b).start()

    @pl.loop(0, KMAX // NBUF)
    def _(kk):
        for i in range(NBUF):
            chunk_body(NBUF * kk + i, i)

    for k in range((KMAX // NBUF) * NBUF, KMAX):
        chunk_body(k, k % NBUF)

    @pl.when(wid == TAIL_W)
    def _():
        base = NFULL * CHUNK
        pltpu.sync_copy(elements_hbm.at[pl.ds(base, TAIL)], tbuf)
        for j in range(TGROUPS):
            elems_g = tbuf[pl.ds(j * 16, 16)]
            idx_g = plsc.load_gather(eidx_v, [elems_g])
            val_g = plsc.load_gather(eye_v, [idx_g, idx_g])
            plsc.store_scatter(tailblk, [idx_g, j * 16 + lane], val_g)
        pltpu.sync_copy(tailblk, out_hbm.at[:, pl.ds(base, TAIL)])

    for k in range(KMAX - NBUF, KMAX):
        @pl.when(wid + NW * k < NFULL)
        def _(k=k):
            blk_out(k, k % NBUF).wait()


def _run(elements, element_idx, eye):
    mesh = plsc.VectorSubcoreMesh(
        core_axis_name="c", subcore_axis_name="s", num_cores=NC, num_subcores=NS
    )
    run = pl.kernel(
        _body,
        out_type=jax.ShapeDtypeStruct((N_ELEM, N_TOK), jnp.float32),
        mesh=mesh,
        scratch_types=[
            pltpu.VMEM((128,), jnp.int32),
            pltpu.VMEM((N_ELEM, N_ELEM), jnp.float32),
            pltpu.VMEM((CHUNK,), jnp.int32),
            pltpu.VMEM((CHUNK,), jnp.int32),
            pltpu.VMEM((CHUNK,), jnp.int32),
            pltpu.VMEM((CHUNK,), jnp.int32),
            pltpu.VMEM((TAIL,), jnp.int32),
            pltpu.VMEM((4, GROUPS, 16), jnp.int32),
            pltpu.VMEM((N_ELEM, CHUNK), jnp.float32),
            pltpu.VMEM((N_ELEM, CHUNK), jnp.float32),
            pltpu.VMEM((N_ELEM, CHUNK), jnp.float32),
            pltpu.VMEM((N_ELEM, CHUNK), jnp.float32),
            pltpu.VMEM((N_ELEM, TAIL), jnp.float32),
            pltpu.SemaphoreType.DMA,
            pltpu.SemaphoreType.DMA,
            pltpu.SemaphoreType.DMA,
            pltpu.SemaphoreType.DMA,
            pltpu.SemaphoreType.DMA,
            pltpu.SemaphoreType.DMA,
            pltpu.SemaphoreType.DMA,
            pltpu.SemaphoreType.DMA,
        ],
        compiler_params=pltpu.CompilerParams(needs_layout_passes=False),
    )
    return run(elements, element_idx, eye)


@jax.jit
def kernel(elements, element_idx, eye):
    return _run(elements, element_idx, eye).T


# speedup vs baseline: 1.0262x; 1.0262x over previous
import jax
import jax.numpy as jnp
from jax import lax
from jax.experimental import pallas as pl
from jax.experimental.pallas import tpu as pltpu, tpu_sc as plsc

N_TOK = 100000
N_ELEM = 100
N_ANUM = 120
NC, NS = 2, 16
NW = NC * NS
CHUNK = 128
NFULL = N_TOK // CHUNK
TAIL = N_TOK - NFULL * CHUNK
KMAX = -(-NFULL // NW)
GROUPS = CHUNK // 16
TGROUPS = TAIL // 16
TAIL_W = NFULL - (KMAX - 1) * NW


def _body(elements_hbm, eidx_hbm, eye_hbm, out_hbm,
          eidx_v, eye_v, ebuf0, ebuf1, tbuf, idxs, blk0, blk1, tailblk,
          esem0, esem1, osem0, osem1):
    ebuf = (ebuf0, ebuf1)
    blk = (blk0, blk1)
    cid = lax.axis_index("c")
    sid = lax.axis_index("s")
    wid = sid * NC + cid

    pltpu.sync_copy(eidx_hbm, eidx_v.at[pl.ds(0, N_ANUM)])
    pltpu.sync_copy(eye_hbm, eye_v)

    zeros16 = jnp.zeros((16,), jnp.float32)

    for buf, width in ((blk0, CHUNK), (blk1, CHUNK), (tailblk, TAIL)):
        @pl.loop(0, N_ELEM)
        def _(r, buf=buf, width=width):
            for c in range(0, width, 16):
                buf[r, pl.ds(c, 16)] = zeros16

    lane = lax.broadcasted_iota(jnp.int32, (16,), 0)
    esem = (esem0, esem1)
    osem = (osem0, osem1)

    def elems_in(k, b):
        base = (wid + NW * k) * CHUNK
        return pltpu.make_async_copy(
            elements_hbm.at[pl.ds(base, CHUNK)], ebuf[b], esem[b]
        )

    def blk_out(k, b):
        base = (wid + NW * k) * CHUNK
        return pltpu.make_async_copy(
            blk[b], out_hbm.at[:, pl.ds(base, CHUNK)], osem[b]
        )

    elems_in(0, 0).start()
    elems_in(1, 1).start()

    def chunk_body(k, b):
        g = wid + NW * k

        @pl.when(k >= 2)
        def _():
            blk_out(k - 2, b).wait()
            for j in range(GROUPS):
                idx_g = idxs[b, j, :]
                plsc.store_scatter(blk[b], [idx_g, j * 16 + lane], zeros16)

        @pl.when(g < NFULL)
        def _():
            elems_in(k, b).wait()
            for j in range(GROUPS):
                elems_g = ebuf[b][pl.ds(j * 16, 16)]
                idx_g = plsc.load_gather(eidx_v, [elems_g])
                val_g = plsc.load_gather(eye_v, [idx_g, idx_g])
                plsc.store_scatter(blk[b], [idx_g, j * 16 + lane], val_g)
                idxs[b, j, :] = idx_g
            blk_out(k, b).start()

            @pl.when(g + 2 * NW < NFULL)
            def _():
                elems_in(k + 2, b).start()

    @pl.loop(0, KMAX // 2)
    def _(kk):
        chunk_body(2 * kk, 0)
        chunk_body(2 * kk + 1, 1)

    chunk_body(KMAX - 1, (KMAX - 1) % 2)

    @pl.when(wid == TAIL_W)
    def _():
        base = NFULL * CHUNK
        pltpu.sync_copy(elements_hbm.at[pl.ds(base, TAIL)], tbuf)
        for j in range(TGROUPS):
            elems_g = tbuf[pl.ds(j * 16, 16)]
            idx_g = plsc.load_gather(eidx_v, [elems_g])
            val_g = plsc.load_gather(eye_v, [idx_g, idx_g])
            plsc.store_scatter(tailblk, [idx_g, j * 16 + lane], val_g)
        pltpu.sync_copy(tailblk, out_hbm.at[:, pl.ds(base, TAIL)])

    for k in (KMAX - 2, KMAX - 1):
        @pl.when(wid + NW * k < NFULL)
        def _(k=k):
            blk_out(k, k % 2).wait()


def _run(elements, element_idx, eye):
    mesh = plsc.VectorSubcoreMesh(
        core_axis_name="c", subcore_axis_name="s", num_cores=NC, num_subcores=NS
    )
    run = pl.kernel(
        _body,
        out_type=jax.ShapeDtypeStruct((N_ELEM, N_TOK), jnp.float32),
        mesh=mesh,
        scratch_types=[
            pltpu.VMEM((128,), jnp.int32),
            pltpu.VMEM((N_ELEM, N_ELEM), jnp.float32),
            pltpu.VMEM((CHUNK,), jnp.int32),
            pltpu.VMEM((CHUNK,), jnp.int32),
            pltpu.VMEM((TAIL,), jnp.int32),
            pltpu.VMEM((2, GROUPS, 16), jnp.int32),
            pltpu.VMEM((N_ELEM, CHUNK), jnp.float32),
            pltpu.VMEM((N_ELEM, CHUNK), jnp.float32),
            pltpu.VMEM((N_ELEM, TAIL), jnp.float32),
            pltpu.SemaphoreType.DMA,
            pltpu.SemaphoreType.DMA,
            pltpu.SemaphoreType.DMA,
            pltpu.SemaphoreType.DMA,
        ],
        compiler_params=pltpu.CompilerParams(needs_layout_passes=False),
    )
    return run(elements, element_idx, eye)


@jax.jit
def kernel(elements, element_idx, eye):
    return _run(elements, element_idx, eye).T
